# baseline (speedup 1.0000x reference)
import functools

import jax
import jax.numpy as jnp
from jax import lax
from jax.experimental import pallas as pl
from jax.experimental.pallas import tpu as pltpu
from jax.experimental.pallas import tpu_sc as plsc

B = 4096
L = 200
D = 64
N = B * L

_info = plsc.get_sparse_core_info()
NC = _info.num_cores
NS = _info.num_subcores
NW = NC * NS
PER_W = N // NW
CHUNK = 640
NBUF = 2
NCHUNK = PER_W // CHUNK
NOUT = NCHUNK // NBUF

_mesh = plsc.VectorSubcoreMesh(core_axis_name="c", subcore_axis_name="s")


@functools.partial(
    pl.kernel,
    mesh=_mesh,
    out_type=jax.ShapeDtypeStruct((N, D), jnp.float32),
    scratch_types=[
        pltpu.VMEM((PER_W,), jnp.int32),
        pltpu.VMEM((NBUF, CHUNK, D), jnp.float32),
        pltpu.SemaphoreType.DMA,
        pltpu.SemaphoreType.DMA,
        pltpu.SemaphoreType.DMA,
        pltpu.SemaphoreType.DMA,
    ],
    compiler_params=pltpu.CompilerParams(use_tc_tiling_on_sc=False),
)
def _gather_kernel(idx_hbm, table_hbm, out_hbm, idx_v, rows_v, g0, g1, s0, s1):
    wid = lax.axis_index("s") * NC + lax.axis_index("c")
    base = wid * PER_W
    gsem = (g0, g1)
    ssem = (s0, s1)

    pltpu.sync_copy(idx_hbm.at[pl.ds(base, PER_W)], idx_v)

    def gather(i, b):
        return pltpu.make_async_copy(
            table_hbm.at[idx_v.at[pl.ds(i * CHUNK, CHUNK)]],
            rows_v.at[b], gsem[b])

    def store(i, b):
        return pltpu.make_async_copy(
            rows_v.at[b], out_hbm.at[pl.ds(base + i * CHUNK, CHUNK)],
            ssem[b])

    for b in range(NBUF):
        gather(b, b).start()

    def body(g, carry):
        i0 = g * NBUF
        for b in range(NBUF):
            gather(i0 + b, b).wait()
            store(i0 + b, b).start()

        @pl.when(g < NOUT - 1)
        def _prefetch():
            for b in range(NBUF):
                store(i0 + b, b).wait()
                gather(i0 + NBUF + b, b).start()

        return carry

    lax.fori_loop(0, NOUT, body, 0)

    for b in range(NBUF):
        store((NOUT - 1) * NBUF + b, b).wait()


def kernel(visit_segments, embedding_table):
    idx = visit_segments.reshape(N).astype(jnp.int32)
    out = _gather_kernel(idx, embedding_table)
    return out.reshape(B, L, D)


# speedup vs baseline: 1.3162x; 1.3162x over previous
import functools

import jax
import jax.numpy as jnp
from jax import lax
from jax.experimental import pallas as pl
from jax.experimental.pallas import tpu as pltpu
from jax.experimental.pallas import tpu_sc as plsc

B = 4096
L = 200
D = 64
N = B * L
NUM_ROWS = 1000

_info = plsc.get_sparse_core_info()
NC = _info.num_cores
NS = _info.num_subcores
NW = NC * NS
PER_W = N // NW
CHUNK = 640
NBUF = 2
NCHUNK = PER_W // CHUNK
NOUT = NCHUNK // NBUF

_mesh = plsc.VectorSubcoreMesh(core_axis_name="c", subcore_axis_name="s")


@functools.partial(
    pl.kernel,
    mesh=_mesh,
    out_type=jax.ShapeDtypeStruct((N, D), jnp.float32),
    scratch_types=[
        pltpu.VMEM((PER_W,), jnp.int32),
        pltpu.VMEM((NBUF, CHUNK, D), jnp.float32),
        pltpu.VMEM_SHARED((NUM_ROWS, D), jnp.float32),
        pltpu.SemaphoreType.DMA,
        pltpu.SemaphoreType.DMA,
        pltpu.SemaphoreType.DMA,
        pltpu.SemaphoreType.DMA,
    ],
    compiler_params=pltpu.CompilerParams(use_tc_tiling_on_sc=False),
)
def _gather_kernel(idx_hbm, table_hbm, out_hbm, idx_v, rows_v, table_s,
                   g0, g1, s0, s1):
    sid = lax.axis_index("s")
    wid = sid * NC + lax.axis_index("c")
    base = wid * PER_W
    gsem = (g0, g1)
    ssem = (s0, s1)

    @pl.when(sid == 0)
    def _stage_table():
        pltpu.sync_copy(table_hbm, table_s)

    pltpu.sync_copy(idx_hbm.at[pl.ds(base, PER_W)], idx_v)
    plsc.subcore_barrier()

    def gather(i, b):
        return pltpu.make_async_copy(
            table_s.at[idx_v.at[pl.ds(i * CHUNK, CHUNK)]],
            rows_v.at[b], gsem[b])

    def store(i, b):
        return pltpu.make_async_copy(
            rows_v.at[b], out_hbm.at[pl.ds(base + i * CHUNK, CHUNK)],
            ssem[b])

    for b in range(NBUF):
        gather(b, b).start()

    def body(g, carry):
        i0 = g * NBUF
        for b in range(NBUF):
            gather(i0 + b, b).wait()
            store(i0 + b, b).start()

        @pl.when(g < NOUT - 1)
        def _prefetch():
            for b in range(NBUF):
                store(i0 + b, b).wait()
                gather(i0 + NBUF + b, b).start()

        return carry

    lax.fori_loop(0, NOUT, body, 0)

    for b in range(NBUF):
        store((NOUT - 1) * NBUF + b, b).wait()


def kernel(visit_segments, embedding_table):
    idx = visit_segments.reshape(N).astype(jnp.int32)
    out = _gather_kernel(idx, embedding_table)
    return out.reshape(B, L, D)
